# baseline (speedup 1.0000x reference)
import functools

import jax
import jax.numpy as jnp
from jax import lax
from jax.experimental import pallas as pl
from jax.experimental.pallas import tpu as pltpu
from jax.experimental.pallas import tpu_sc as plsc


def _gather3(embedding, kgg_table, relation_table, scg_ids, relation_ids,
             kgg_ids):
    B = scg_ids.shape[0]
    D = embedding.shape[1]
    NC, NS = 2, 16
    NW = NC * NS
    b_per_w = B // NW
    mesh = plsc.VectorSubcoreMesh(core_axis_name="c", subcore_axis_name="s")

    @functools.partial(
        pl.kernel,
        mesh=mesh,
        compiler_params=pltpu.CompilerParams(use_tc_tiling_on_sc=False),
        out_type=(
            jax.ShapeDtypeStruct((B, D), jnp.float32),
            jax.ShapeDtypeStruct((B, D), jnp.float32),
            jax.ShapeDtypeStruct((B, D), jnp.float32),
        ),
        scratch_types=[
            pltpu.VMEM((b_per_w,), jnp.int32),
            pltpu.VMEM((b_per_w,), jnp.int32),
            pltpu.VMEM((b_per_w,), jnp.int32),
            pltpu.VMEM((b_per_w, D), jnp.float32),
            pltpu.VMEM((b_per_w, D), jnp.float32),
            pltpu.VMEM((b_per_w, D), jnp.float32),
            pltpu.SemaphoreType.DMA,
            pltpu.SemaphoreType.DMA,
            pltpu.SemaphoreType.DMA,
            pltpu.SemaphoreType.DMA,
        ],
    )
    def k(emb_hbm, kgg_hbm, rel_hbm, scg_hbm, relid_hbm, kggid_hbm,
          out_scg, out_kgg, out_rel,
          idx_scg, idx_kgg, idx_rel, rows_scg, rows_kgg, rows_rel,
          sem_scg, sem_kgg, sem_rel, sem_out):
        wid = lax.axis_index("s") * NC + lax.axis_index("c")
        base = wid * b_per_w
        pltpu.sync_copy(scg_hbm.at[pl.ds(base, b_per_w)], idx_scg)
        pltpu.sync_copy(kggid_hbm.at[pl.ds(base, b_per_w)], idx_kgg)
        pltpu.sync_copy(relid_hbm.at[pl.ds(base, b_per_w)], idx_rel)
        g1 = pltpu.async_copy(emb_hbm.at[idx_scg], rows_scg, sem_scg)
        g2 = pltpu.async_copy(kgg_hbm.at[idx_kgg], rows_kgg, sem_kgg)
        g3 = pltpu.async_copy(rel_hbm.at[idx_rel], rows_rel, sem_rel)
        g1.wait()
        w1 = pltpu.async_copy(rows_scg, out_scg.at[pl.ds(base, b_per_w)],
                              sem_out)
        g2.wait()
        w2 = pltpu.async_copy(rows_kgg, out_kgg.at[pl.ds(base, b_per_w)],
                              sem_out)
        g3.wait()
        w3 = pltpu.async_copy(rows_rel, out_rel.at[pl.ds(base, b_per_w)],
                              sem_out)
        w1.wait()
        w2.wait()
        w3.wait()

    return k(embedding, kgg_table, relation_table, scg_ids, relation_ids,
             kgg_ids)


def kernel(embedding, kgg_table, relation_table, scg_ids, relation_ids,
           kgg_ids):
    return _gather3(embedding, kgg_table, relation_table,
                    scg_ids.astype(jnp.int32), relation_ids.astype(jnp.int32),
                    kgg_ids.astype(jnp.int32))
